# baseline (speedup 1.0000x reference)
import functools

import jax
import jax.numpy as jnp
from jax import lax
from jax.experimental import pallas as pl
from jax.experimental.pallas import tpu as pltpu
from jax.experimental.pallas import tpu_sc as plsc

B, S, D = 4, 8192, 768
K = 1024
N = B * S
NC, NS = 2, 16
BK1 = 4096
BK4 = 2048
CH = K // 8
MININT = -2**31


def _threshold(wfull, thr_ref, pidx_ref):
    w = wfull + 0.0
    bits = jax.lax.bitcast_convert_type(w, jnp.int32)
    skey = jnp.where(bits >= 0, bits, bits ^ jnp.int32(0x7FFFFFFF))
    minint = jnp.int32(MININT)
    prefix = jnp.zeros((B, 1, 1), jnp.int32)
    for t in range(31, -1, -1):
        bit = minint if t == 31 else jnp.int32(1 << t)
        cand = prefix | bit
        cand_s = cand ^ minint
        cnt = jnp.sum((skey >= cand_s).astype(jnp.int32), axis=(1, 2),
                      keepdims=True)
        prefix = jnp.where(cnt >= K, cand, prefix)
    ks = prefix ^ minint
    tbits = jnp.where(ks >= 0, ks, ks ^ jnp.int32(0x7FFFFFFF))
    thr = jax.lax.bitcast_convert_type(tbits, jnp.float32)
    thr_ref[...] = jnp.broadcast_to(thr.reshape(B, 1), (B, 128))
    sidx = (jax.lax.broadcasted_iota(jnp.int32, (B, S // 128, 128), 1) * 128
            + jax.lax.broadcasted_iota(jnp.int32, (B, S // 128, 128), 2))
    big = jnp.where(skey == ks, sidx, jnp.int32(2 ** 30))
    ploc = jnp.min(big, axis=(1, 2), keepdims=True).reshape(B, 1)
    pglob = ploc + jax.lax.broadcasted_iota(jnp.int32, (B, 1), 0) * S
    pidx_ref[...] = jnp.broadcast_to(pglob, (B, 128))


def _k1_body(x_ref, wr_ref, br_ref, w_ref, out_ref, thr_ref, pidx_ref):
    b = pl.program_id(0)
    j = pl.program_id(1)
    xb = x_ref[...]
    w = jax.lax.dot_general(xb, wr_ref[...], (((1,), (0,)), ((), ())),
                            preferred_element_type=jnp.float32)
    w = w + br_ref[0]
    w_ref[pl.ds(b, 1), pl.ds(j * (BK1 // 128), BK1 // 128), :] = (
        w.reshape(1, BK1 // 128, 128))
    out_ref[...] = xb

    @pl.when((b == B - 1) & (j == S // BK1 - 1))
    def _last():
        _threshold(w_ref[...], thr_ref, pidx_ref)


def _run_k1(x_flat, Wr, br):
    grid = (B, S // BK1)
    return pl.pallas_call(
        _k1_body,
        grid=grid,
        in_specs=[
            pl.BlockSpec((BK1, D), lambda b, j: (b * (S // BK1) + j, 0)),
            pl.BlockSpec((D, 1), lambda b, j: (0, 0)),
            pl.BlockSpec(memory_space=pltpu.SMEM),
        ],
        out_specs=[
            pl.BlockSpec((B, S // 128, 128), lambda b, j: (0, 0, 0)),
            pl.BlockSpec((BK1, D), lambda b, j: (b * (S // BK1) + j, 0)),
            pl.BlockSpec((B, 128), lambda b, j: (0, 0)),
            pl.BlockSpec((B, 128), lambda b, j: (0, 0)),
        ],
        out_shape=[
            jax.ShapeDtypeStruct((B, S // 128, 128), jnp.float32),
            jax.ShapeDtypeStruct((N, D), jnp.float32),
            jax.ShapeDtypeStruct((B, 128), jnp.float32),
            jax.ShapeDtypeStruct((B, 128), jnp.int32),
        ],
    )(x_flat, Wr, br)


OC = S // 8


def _k3_body(w_hbm, thr_hbm, pidx_hbm, x_hbm, idx_out, wg_out,
             xg_out, loc_idx, loc_wg, loc_cnt, w_v, thr_v, pidx_v,
             idxb_v, wgb_v, cntv_v, peer_idx_v, peer_wg_v, idxg_v,
             xg_v, sem):
    c = lax.axis_index("c")
    s = lax.axis_index("s")
    r = c * 2 + s // 8
    o = s % 8

    pltpu.sync_copy(w_hbm.at[r, pl.ds(o * OC, OC)], w_v)
    pltpu.sync_copy(thr_hbm.at[r, pl.ds(0, 16)], thr_v)
    pltpu.sync_copy(pidx_hbm.at[r, pl.ds(0, 16)], pidx_v)
    thrv = thr_v[...]
    lanes = lax.iota(jnp.int32, 16)
    base0 = r * S + o * OC

    @pl.loop(0, OC // 16, init_carry=jnp.int32(0))
    def _scan(j, cnt):
        wv = w_v[pl.ds(j * 16, 16)]
        m = wv > thrv
        iv = base0 + j * 16 + lanes
        plsc.store_compressed(idxb_v.at[pl.ds(cnt, 16)], iv, mask=m)
        plsc.store_compressed(wgb_v.at[pl.ds(cnt, 16)], wv, mask=m)
        pc = plsc.all_reduce_population_count(m)
        return cnt + pc[0]

    cntv_v[...] = jnp.broadcast_to(_scan, (16,))
    pltpu.sync_copy(cntv_v, loc_cnt.at[r, o])
    pltpu.sync_copy(idxb_v, loc_idx.at[r, o])
    pltpu.sync_copy(wgb_v, loc_wg.at[r, o])

    plsc.subcore_barrier()

    @pl.when(o == 0)
    def _merge():
        cnt0 = _scan
        cnt = cnt0

        for o2 in range(1, 8):
            pltpu.sync_copy(loc_idx.at[r, o2], peer_idx_v)
            pltpu.sync_copy(loc_wg.at[r, o2], peer_wg_v)
            pltpu.sync_copy(loc_cnt.at[r, o2], cntv_v)
            cnt_o = cntv_v[...][0]

            def _mbody(j, cc):
                vi = peer_idx_v[pl.ds(j * 16, 16)]
                vw = peer_wg_v[pl.ds(j * 16, 16)]
                m = (j * 16 + lanes) < cnt_o
                plsc.store_compressed(idxb_v.at[pl.ds(cc, 16)], vi, mask=m)
                plsc.store_compressed(wgb_v.at[pl.ds(cc, 16)], vw, mask=m)
                pc = plsc.all_reduce_population_count(m)
                return cc + pc[0]

            cnt = lax.fori_loop(0, (cnt_o + 15) // 16, _mbody, cnt)

        padv = pidx_v[...]
        nanv = jnp.full((16,), float("nan"), jnp.float32)
        cntv = jnp.broadcast_to(cnt, (16,))

        @pl.loop(0, K // 16)
        def _pad(i):
            pos = i * 16 + lanes
            m = pos >= cntv
            plsc.store_scatter(idxb_v, [pos], padv, mask=m)
            plsc.store_scatter(wgb_v, [pos], nanv, mask=m)

        pltpu.sync_copy(idxb_v, idx_out.at[r])
        pltpu.sync_copy(wgb_v, wg_out.at[r])

    plsc.subcore_barrier()

    pltpu.sync_copy(idx_out.at[r, pl.ds(o * CH, CH)], idxg_v)
    pltpu.async_copy(x_hbm.at[idxg_v], xg_v, sem).wait()
    pltpu.sync_copy(xg_v, xg_out.at[pl.ds(r * K + o * CH, CH)])


def _run_k3(weights, thr, pidx, x_flat):
    mesh = plsc.VectorSubcoreMesh(core_axis_name="c", subcore_axis_name="s",
                                  num_cores=NC, num_subcores=NS)
    kfn = pl.kernel(
        _k3_body,
        out_type=[
            jax.ShapeDtypeStruct((B, K), jnp.int32),
            jax.ShapeDtypeStruct((B, K), jnp.float32),
            jax.ShapeDtypeStruct((B * K, D), jnp.float32),
            jax.ShapeDtypeStruct((B, 8, K), jnp.int32),
            jax.ShapeDtypeStruct((B, 8, K), jnp.float32),
            jax.ShapeDtypeStruct((B, 8, 16), jnp.int32),
        ],
        mesh=mesh,
        compiler_params=pltpu.CompilerParams(needs_layout_passes=False),
        scratch_types=[
            pltpu.VMEM((OC,), jnp.float32),
            pltpu.VMEM((16,), jnp.float32),
            pltpu.VMEM((16,), jnp.int32),
            pltpu.VMEM((K,), jnp.int32),
            pltpu.VMEM((K,), jnp.float32),
            pltpu.VMEM((16,), jnp.int32),
            pltpu.VMEM((K,), jnp.int32),
            pltpu.VMEM((K,), jnp.float32),
            pltpu.VMEM((CH,), jnp.int32),
            pltpu.VMEM((CH, D), jnp.float32),
            pltpu.SemaphoreType.DMA,
        ],
    )
    idx, wg, xg, _, _, _ = kfn(weights, thr, pidx, x_flat)
    return idx, wg, xg


def _k4_body(xg_ref, wb_ref, bb_ref, wg_ref, yc_ref):
    xg = xg_ref[...]
    y = jax.lax.dot_general(xg, wb_ref[...], (((1,), (0,)), ((), ())),
                            preferred_element_type=jnp.float32)
    y = jnp.tanh(y + bb_ref[...])
    prod = y * wg_ref[...]
    yc_ref[...] = jnp.where(prod != prod, xg, prod)


def _run_k4(xg, Wb, bb, wg_col):
    grid = (B * K // BK4,)
    return pl.pallas_call(
        _k4_body,
        grid=grid,
        in_specs=[
            pl.BlockSpec((BK4, D), lambda i: (i, 0)),
            pl.BlockSpec((D, D), lambda i: (0, 0)),
            pl.BlockSpec((1, D), lambda i: (0, 0)),
            pl.BlockSpec((BK4, 1), lambda i: (i, 0)),
        ],
        out_specs=pl.BlockSpec((BK4, D), lambda i: (i, 0)),
        out_shape=jax.ShapeDtypeStruct((B * K, D), jnp.float32),
    )(xg, Wb, bb.reshape(1, D), wg_col)


def _k5_body(yc_hbm, idx_hbm, out_ref, idx_v, yc_v, sem):
    c = lax.axis_index("c")
    s = lax.axis_index("s")
    r = c * 2 + s // 8
    ch = s % 8
    pltpu.sync_copy(idx_hbm.at[r, pl.ds(ch * CH, CH)], idx_v)
    pltpu.sync_copy(yc_hbm.at[pl.ds(r * K + ch * CH, CH)], yc_v)
    pltpu.async_copy(yc_v, out_ref.at[idx_v], sem).wait()


def _run_k5(yc, idx, out_ref):
    mesh = plsc.VectorSubcoreMesh(core_axis_name="c", subcore_axis_name="s",
                                  num_cores=NC, num_subcores=NS)
    kfn = pl.kernel(
        _k5_body,
        out_type=(),
        mesh=mesh,
        compiler_params=pltpu.CompilerParams(needs_layout_passes=False),
        scratch_types=[
            pltpu.VMEM((CH,), jnp.int32),
            pltpu.VMEM((CH, D), jnp.float32),
            pltpu.SemaphoreType.DMA,
        ],
    )
    kfn(yc, idx, out_ref)


def kernel(x, causal_mask, position_ids, cache_position, Wr, br, Wb, bb):
    x_flat = x.reshape(N, D)
    weights, out_full, thr, pidx = _run_k1(x_flat, Wr, br)
    idx, wg, xg = _run_k3(weights.reshape(B, S), thr, pidx, x_flat)
    yc = _run_k4(xg, Wb, bb, wg.reshape(B * K, 1))
    out_r = jax.new_ref(out_full)
    _run_k5(yc, idx, out_r)
    out = out_r[...]
    return out.reshape(B, S, D)


# speedup vs baseline: 1.0656x; 1.0656x over previous
import jax
import jax.numpy as jnp
from jax import lax
from jax.experimental import pallas as pl
from jax.experimental.pallas import tpu as pltpu
from jax.experimental.pallas import tpu_sc as plsc

B, S, D = 4, 8192, 768
K = 1024
N = B * S
NC, NS = 2, 16
BK1 = 4096
BK4 = 2048
CH = K // 8
MININT = -2**31


def _threshold(wfull, thr_ref, pidx_ref):
    w = wfull + 0.0
    bits = jax.lax.bitcast_convert_type(w, jnp.int32)
    skey = jnp.where(bits >= 0, bits, bits ^ jnp.int32(0x7FFFFFFF))
    minint = jnp.int32(MININT)
    prefix = jnp.zeros((B, 1, 1), jnp.int32)
    for t in range(31, -1, -1):
        bit = minint if t == 31 else jnp.int32(1 << t)
        cand = prefix | bit
        cand_s = cand ^ minint
        cnt = jnp.sum((skey >= cand_s).astype(jnp.int32), axis=(1, 2),
                      keepdims=True)
        prefix = jnp.where(cnt >= K, cand, prefix)
    ks = prefix ^ minint
    tbits = jnp.where(ks >= 0, ks, ks ^ jnp.int32(0x7FFFFFFF))
    thr = jax.lax.bitcast_convert_type(tbits, jnp.float32)
    thr_ref[...] = jnp.broadcast_to(thr.reshape(B, 1), (B, 128))
    sidx = (jax.lax.broadcasted_iota(jnp.int32, (B, S // 128, 128), 1) * 128
            + jax.lax.broadcasted_iota(jnp.int32, (B, S // 128, 128), 2))
    big = jnp.where(skey == ks, sidx, jnp.int32(2 ** 30))
    ploc = jnp.min(big, axis=(1, 2), keepdims=True).reshape(B, 1)
    pglob = ploc + jax.lax.broadcasted_iota(jnp.int32, (B, 1), 0) * S
    pidx_ref[...] = jnp.broadcast_to(pglob, (B, 128))


def _k1_body(x_ref, wr_ref, br_ref, w_ref, out_ref, thr_ref, pidx_ref):
    b = pl.program_id(0)
    j = pl.program_id(1)
    xb = x_ref[...]
    w = jax.lax.dot_general(xb, wr_ref[...], (((1,), (0,)), ((), ())),
                            preferred_element_type=jnp.float32)
    w = w + br_ref[0]
    w_ref[pl.ds(b, 1), pl.ds(j * (BK1 // 128), BK1 // 128), :] = (
        w.reshape(1, BK1 // 128, 128))
    out_ref[...] = xb

    @pl.when((b == B - 1) & (j == S // BK1 - 1))
    def _last():
        _threshold(w_ref[...], thr_ref, pidx_ref)


def _run_k1(x_flat, Wr, br):
    grid = (B, S // BK1)
    return pl.pallas_call(
        _k1_body,
        grid=grid,
        in_specs=[
            pl.BlockSpec((BK1, D), lambda b, j: (b * (S // BK1) + j, 0)),
            pl.BlockSpec((D, 1), lambda b, j: (0, 0)),
            pl.BlockSpec(memory_space=pltpu.SMEM),
        ],
        out_specs=[
            pl.BlockSpec((B, S // 128, 128), lambda b, j: (0, 0, 0)),
            pl.BlockSpec((BK1, D), lambda b, j: (b * (S // BK1) + j, 0)),
            pl.BlockSpec((B, 128), lambda b, j: (0, 0)),
            pl.BlockSpec((B, 128), lambda b, j: (0, 0)),
        ],
        out_shape=[
            jax.ShapeDtypeStruct((B, S // 128, 128), jnp.float32),
            jax.ShapeDtypeStruct((N, D), jnp.float32),
            jax.ShapeDtypeStruct((B, 128), jnp.float32),
            jax.ShapeDtypeStruct((B, 128), jnp.int32),
        ],
    )(x_flat, Wr, br)


def _k3_body(w_hbm, thr_hbm, pidx_hbm, x_hbm, idx_out, wg_out,
             xg_out, w_v, thr_v, pidx_v, idxb_v, wgb_v, idxg_v,
             xg_v, sem):
    c = lax.axis_index("c")
    s = lax.axis_index("s")

    @pl.when(s < 2)
    def _compact():
        r = c * 2 + s
        pltpu.sync_copy(w_hbm.at[r], w_v)
        pltpu.sync_copy(thr_hbm.at[r, pl.ds(0, 16)], thr_v)
        pltpu.sync_copy(pidx_hbm.at[r, pl.ds(0, 16)], pidx_v)
        thrv = thr_v[...]
        padv = pidx_v[...]
        nanv = jnp.full((16,), float("nan"), jnp.float32)

        @pl.loop(0, K // 16)
        def _init(i):
            idxb_v[pl.ds(i * 16, 16)] = padv
            wgb_v[pl.ds(i * 16, 16)] = nanv

        lanes = lax.iota(jnp.int32, 16)
        base0 = r * S

        @pl.loop(0, S // 16, init_carry=jnp.int32(0))
        def _scan(j, cnt):
            wv = w_v[pl.ds(j * 16, 16)]
            m = wv > thrv
            iv = base0 + j * 16 + lanes
            plsc.store_compressed(idxb_v.at[pl.ds(cnt, 16)], iv, mask=m)
            plsc.store_compressed(wgb_v.at[pl.ds(cnt, 16)], wv, mask=m)
            pc = plsc.all_reduce_population_count(m)
            return cnt + pc[0]

        pltpu.sync_copy(idxb_v, idx_out.at[r])
        pltpu.sync_copy(wgb_v, wg_out.at[r])

    plsc.subcore_barrier()

    r = c * 2 + s // 8
    ch = s % 8
    pltpu.sync_copy(idx_out.at[r, pl.ds(ch * CH, CH)], idxg_v)
    pltpu.async_copy(x_hbm.at[idxg_v], xg_v, sem).wait()
    pltpu.sync_copy(xg_v, xg_out.at[pl.ds(r * K + ch * CH, CH)])


def _run_k3(weights, thr, pidx, x_flat):
    mesh = plsc.VectorSubcoreMesh(core_axis_name="c", subcore_axis_name="s",
                                  num_cores=NC, num_subcores=NS)
    kfn = pl.kernel(
        _k3_body,
        out_type=[
            jax.ShapeDtypeStruct((B, K), jnp.int32),
            jax.ShapeDtypeStruct((B, K), jnp.float32),
            jax.ShapeDtypeStruct((B * K, D), jnp.float32),
        ],
        mesh=mesh,
        compiler_params=pltpu.CompilerParams(needs_layout_passes=False),
        scratch_types=[
            pltpu.VMEM((S,), jnp.float32),
            pltpu.VMEM((16,), jnp.float32),
            pltpu.VMEM((16,), jnp.int32),
            pltpu.VMEM((K,), jnp.int32),
            pltpu.VMEM((K,), jnp.float32),
            pltpu.VMEM((CH,), jnp.int32),
            pltpu.VMEM((CH, D), jnp.float32),
            pltpu.SemaphoreType.DMA,
        ],
    )
    return kfn(weights, thr, pidx, x_flat)


def _k4_body(xg_ref, wb_ref, bb_ref, wg_ref, yc_ref):
    xg = xg_ref[...]
    y = jax.lax.dot_general(xg, wb_ref[...], (((1,), (0,)), ((), ())),
                            preferred_element_type=jnp.float32)
    y = jnp.tanh(y + bb_ref[...])
    prod = y * wg_ref[...]
    yc_ref[...] = jnp.where(prod != prod, xg, prod)


def _run_k4(xg, Wb, bb, wg_col):
    grid = (B * K // BK4,)
    return pl.pallas_call(
        _k4_body,
        grid=grid,
        in_specs=[
            pl.BlockSpec((BK4, D), lambda i: (i, 0)),
            pl.BlockSpec((D, D), lambda i: (0, 0)),
            pl.BlockSpec((1, D), lambda i: (0, 0)),
            pl.BlockSpec((BK4, 1), lambda i: (i, 0)),
        ],
        out_specs=pl.BlockSpec((BK4, D), lambda i: (i, 0)),
        out_shape=jax.ShapeDtypeStruct((B * K, D), jnp.float32),
    )(xg, Wb, bb.reshape(1, D), wg_col)


def _k5_body(yc_hbm, idx_hbm, out_ref, idx_v, yc_v, sem):
    c = lax.axis_index("c")
    s = lax.axis_index("s")
    r = c * 2 + s // 8
    ch = s % 8
    pltpu.sync_copy(idx_hbm.at[r, pl.ds(ch * CH, CH)], idx_v)
    pltpu.sync_copy(yc_hbm.at[pl.ds(r * K + ch * CH, CH)], yc_v)
    pltpu.async_copy(yc_v, out_ref.at[idx_v], sem).wait()


def _run_k5(yc, idx, out_ref):
    mesh = plsc.VectorSubcoreMesh(core_axis_name="c", subcore_axis_name="s",
                                  num_cores=NC, num_subcores=NS)
    kfn = pl.kernel(
        _k5_body,
        out_type=(),
        mesh=mesh,
        compiler_params=pltpu.CompilerParams(needs_layout_passes=False),
        scratch_types=[
            pltpu.VMEM((CH,), jnp.int32),
            pltpu.VMEM((CH, D), jnp.float32),
            pltpu.SemaphoreType.DMA,
        ],
    )
    kfn(yc, idx, out_ref)


def kernel(x, causal_mask, position_ids, cache_position, Wr, br, Wb, bb):
    x_flat = x.reshape(N, D)
    weights, out_full, thr, pidx = _run_k1(x_flat, Wr, br)
    idx, wg, xg = _run_k3(weights.reshape(B, S), thr, pidx, x_flat)
    yc = _run_k4(xg, Wb, bb, wg.reshape(B * K, 1))
    out_r = jax.new_ref(out_full)
    _run_k5(yc, idx, out_r)
    out = out_r[...]
    return out.reshape(B, S, D)
